# Initial kernel scaffold; baseline (speedup 1.0000x reference)
import jax
import jax.numpy as jnp
from jax.experimental import pallas as pl


def kernel(features, xy, graph_ids):
    raise NotImplementedError("write your pallas kernel here")


# speedup vs baseline: 2.0656x; 2.0656x over previous
import functools

import jax
import jax.numpy as jnp
from jax import lax
from jax.experimental import pallas as pl
from jax.experimental.pallas import tpu as pltpu
from jax.experimental.pallas import tpu_sc as plsc

N_GRAPHS = 16
GRID = 8
D = 128
N_NODES = 100000

NW = 32
CHUNK = 128
N_CHUNKS = 25
PER_W = CHUNK * N_CHUNKS
N_PAD = NW * PER_W
NBINS = N_GRAPHS * GRID * GRID
ROWS_PER_TILE = NBINS // 16

_mesh = plsc.VectorSubcoreMesh(core_axis_name="c", subcore_axis_name="s")


@functools.partial(
    pl.kernel,
    out_type=jax.ShapeDtypeStruct((2, NBINS, D), jnp.float32),
    mesh=_mesh,
    compiler_params=pltpu.CompilerParams(needs_layout_passes=False),
    scratch_types=[
        pltpu.VMEM((CHUNK, D), jnp.float32),
        pltpu.VMEM((CHUNK,), jnp.int32),
        pltpu.VMEM((CHUNK,), jnp.int32),
        pltpu.VMEM((CHUNK,), jnp.int32),
        pltpu.VMEM((CHUNK,), jnp.float32),
        pltpu.VMEM((CHUNK,), jnp.int32),
        pltpu.VMEM((ROWS_PER_TILE, D), jnp.float32),
        pltpu.VMEM_SHARED((NBINS, D), jnp.float32),
    ],
)
def _spp_scatter(feat, x, y, cnt, gid, out, feat_c, idx_c, x_c, y_c, cnt_c,
                 gid_c, bounce, acc):
    cid = lax.axis_index("c")
    sid = lax.axis_index("s")
    w = cid * 16 + sid
    base_w = w * PER_W

    zero16 = jnp.zeros((16,), jnp.float32)

    def zero_row(r, _):
        for db in range(D // 16):
            bounce[r, pl.ds(db * 16, 16)] = zero16
        return 0

    lax.fori_loop(0, ROWS_PER_TILE, zero_row, 0)
    pltpu.sync_copy(bounce, acc.at[pl.ds(sid * ROWS_PER_TILE, ROWS_PER_TILE)])
    plsc.subcore_barrier()

    def chunk_body(c, _):
        base = base_w + c * CHUNK
        pltpu.sync_copy(feat.at[pl.ds(base, CHUNK)], feat_c)
        pltpu.sync_copy(x.at[pl.ds(base, CHUNK)], x_c)
        pltpu.sync_copy(y.at[pl.ds(base, CHUNK)], y_c)
        pltpu.sync_copy(cnt.at[pl.ds(base, CHUNK)], cnt_c)
        pltpu.sync_copy(gid.at[pl.ds(base, CHUNK)], gid_c)

        def bin_body(i, _):
            sl = pl.ds(i * 16, 16)
            idx_c[sl] = gid_c[sl] * (GRID * GRID) + x_c[sl] * GRID + y_c[sl]
            return 0

        lax.fori_loop(0, CHUNK // 16, bin_body, 0)

        def node_body(j, _):
            cv = plsc.load_gather(cnt_c, [jnp.full((16,), j, jnp.int32)])
            inv = 1.0 / cv
            for db in range(D // 16):
                sl = pl.ds(db * 16, 16)
                feat_c[j, sl] = feat_c[j, sl] * inv
            return 0

        lax.fori_loop(0, CHUNK, node_body, 0)
        pltpu.sync_copy(feat_c, acc.at[idx_c], add=True)
        return 0

    lax.fori_loop(0, N_CHUNKS, chunk_body, 0)
    plsc.subcore_barrier()

    sl = pl.ds(sid * ROWS_PER_TILE, ROWS_PER_TILE)
    pltpu.sync_copy(acc.at[sl], bounce)
    pltpu.sync_copy(bounce, out.at[cid, sl])


def _add_partials_body(p_ref, o_ref):
    o_ref[...] = p_ref[0] + p_ref[1]


_add_partials = pl.pallas_call(
    _add_partials_body,
    out_shape=jax.ShapeDtypeStruct((NBINS, D), jnp.float32),
)


def kernel(features, xy, graph_ids):
    pad = N_PAD - N_NODES
    featp = jnp.concatenate(
        [features, jnp.zeros((pad, D), features.dtype)], axis=0)
    xcol = jnp.concatenate(
        [xy[:, 0].astype(jnp.int32), jnp.zeros((pad,), jnp.int32)])
    ycol = jnp.concatenate(
        [xy[:, 1].astype(jnp.int32), jnp.zeros((pad,), jnp.int32)])
    cntf = jnp.concatenate(
        [xy[:, 2].astype(jnp.float32), jnp.ones((pad,), jnp.float32)])
    gidp = jnp.concatenate(
        [graph_ids.astype(jnp.int32), jnp.zeros((pad,), jnp.int32)])
    partials = _spp_scatter(featp, xcol, ycol, cntf, gidp)
    out = _add_partials(partials)
    return out.reshape(N_GRAPHS, GRID, GRID, D)


# speedup vs baseline: 2.3904x; 1.1572x over previous
import functools

import jax
import jax.numpy as jnp
from jax import lax
from jax.experimental import pallas as pl
from jax.experimental.pallas import tpu as pltpu
from jax.experimental.pallas import tpu_sc as plsc

N_GRAPHS = 16
GRID = 8
D = 128
N_NODES = 100000

NW = 32
CHUNK = 128
N_CHUNKS = 25
PER_W = CHUNK * N_CHUNKS
N_PAD = NW * PER_W
N_ALLOC = N_PAD + CHUNK
NBINS = N_GRAPHS * GRID * GRID
ROWS_PER_TILE = NBINS // 16

_mesh = plsc.VectorSubcoreMesh(core_axis_name="c", subcore_axis_name="s")


@functools.partial(
    pl.kernel,
    out_type=jax.ShapeDtypeStruct((2, NBINS, D), jnp.float32),
    mesh=_mesh,
    compiler_params=pltpu.CompilerParams(needs_layout_passes=False),
    scratch_types=[
        pltpu.VMEM((CHUNK, D), jnp.float32),
        pltpu.VMEM((CHUNK, D), jnp.float32),
        pltpu.VMEM((CHUNK, 4), jnp.float32),
        pltpu.VMEM((CHUNK, 4), jnp.float32),
        pltpu.VMEM((CHUNK,), jnp.int32),
        pltpu.VMEM((CHUNK,), jnp.int32),
        pltpu.VMEM((ROWS_PER_TILE, D), jnp.float32),
        pltpu.VMEM_SHARED((NBINS, D), jnp.float32),
        pltpu.SemaphoreType.DMA,
        pltpu.SemaphoreType.DMA,
        pltpu.SemaphoreType.DMA,
        pltpu.SemaphoreType.DMA,
    ],
)
def _spp_scatter(feat, aux, out, feat_a, feat_b, aux_a, aux_b, idx_a, idx_b,
                 bounce, acc, sem_fa, sem_fb, sem_sa, sem_sb):
    cid = lax.axis_index("c")
    sid = lax.axis_index("s")
    w = cid * 16 + sid
    base_w = w * PER_W

    iota16 = lax.broadcasted_iota(jnp.int32, (16,), 0)
    col0 = jnp.full((16,), 0, jnp.int32)
    col1 = jnp.full((16,), 1, jnp.int32)
    col2 = jnp.full((16,), 2, jnp.int32)
    col3 = jnp.full((16,), 3, jnp.int32)
    zero16 = jnp.zeros((16,), jnp.float32)

    def start_fetch(c, featb, auxb, sem):
        base = base_w + c * CHUNK
        pltpu.async_copy(feat.at[pl.ds(base, CHUNK)], featb, sem)
        pltpu.async_copy(aux.at[pl.ds(base, CHUNK)], auxb, sem)

    def wait_fetch(featb, auxb, sem):
        pltpu.make_async_copy(feat.at[pl.ds(0, CHUNK)], featb, sem).wait()
        pltpu.make_async_copy(aux.at[pl.ds(0, CHUNK)], auxb, sem).wait()

    def process(featb, auxb, idxb):
        def bin_body(i, _):
            rows = iota16 + i * 16
            xv = plsc.load_gather(auxb, [rows, col0])
            yv = plsc.load_gather(auxb, [rows, col1])
            gv = plsc.load_gather(auxb, [rows, col3])
            binv = gv * float(GRID * GRID) + xv * float(GRID) + yv
            idxb[pl.ds(i * 16, 16)] = binv.astype(jnp.int32)
            return 0

        lax.fori_loop(0, CHUNK // 16, bin_body, 0)

        def node_body(j, _):
            cv = plsc.load_gather(auxb, [jnp.full((16,), j, jnp.int32), col2])
            inv = 1.0 / cv
            for db in range(D // 16):
                sl = pl.ds(db * 16, 16)
                featb[j, sl] = featb[j, sl] * inv
            return 0

        lax.fori_loop(0, CHUNK, node_body, 0, unroll=4)

    def zero_row(r, _):
        for db in range(D // 16):
            bounce[r, pl.ds(db * 16, 16)] = zero16
        return 0

    lax.fori_loop(0, ROWS_PER_TILE, zero_row, 0)
    pltpu.sync_copy(bounce, acc.at[pl.ds(sid * ROWS_PER_TILE, ROWS_PER_TILE)])
    plsc.subcore_barrier()

    start_fetch(0, feat_a, aux_a, sem_fa)
    start_fetch(1, feat_b, aux_b, sem_fb)

    def pipe_body(k, _):
        ca = 2 * k
        wait_fetch(feat_a, aux_a, sem_fa)
        process(feat_a, aux_a, idx_a)
        h_sa = pltpu.async_copy(feat_a, acc.at[idx_a], sem_sa, add=True)
        wait_fetch(feat_b, aux_b, sem_fb)
        process(feat_b, aux_b, idx_b)
        h_sa.wait()
        start_fetch(ca + 2, feat_a, aux_a, sem_fa)
        h_sb = pltpu.async_copy(feat_b, acc.at[idx_b], sem_sb, add=True)
        h_sb.wait()
        start_fetch(ca + 3, feat_b, aux_b, sem_fb)
        return 0

    lax.fori_loop(0, (N_CHUNKS - 1) // 2, pipe_body, 0)

    wait_fetch(feat_a, aux_a, sem_fa)
    process(feat_a, aux_a, idx_a)
    pltpu.async_copy(feat_a, acc.at[idx_a], sem_sa, add=True).wait()
    wait_fetch(feat_b, aux_b, sem_fb)

    plsc.subcore_barrier()
    sl = pl.ds(sid * ROWS_PER_TILE, ROWS_PER_TILE)
    pltpu.sync_copy(acc.at[sl], bounce)
    pltpu.sync_copy(bounce, out.at[cid, sl])


def _add_partials_body(p_ref, o_ref):
    o_ref[...] = p_ref[0] + p_ref[1]


_add_partials = pl.pallas_call(
    _add_partials_body,
    out_shape=jax.ShapeDtypeStruct((NBINS, D), jnp.float32),
)


def kernel(features, xy, graph_ids):
    pad = N_ALLOC - N_NODES
    featp = jnp.concatenate(
        [features, jnp.zeros((pad, D), features.dtype)], axis=0)
    aux = jnp.concatenate([
        xy[:, :2].astype(jnp.float32),
        xy[:, 2:3].astype(jnp.float32),
        graph_ids[:, None].astype(jnp.float32),
    ], axis=1)
    aux_pad = jnp.tile(
        jnp.array([[0.0, 0.0, 1.0, 0.0]], jnp.float32), (pad, 1))
    auxp = jnp.concatenate([aux, aux_pad], axis=0)
    partials = _spp_scatter(featp, auxp)
    out = _add_partials(partials)
    return out.reshape(N_GRAPHS, GRID, GRID, D)


# speedup vs baseline: 3.2439x; 1.3570x over previous
import functools

import jax
import jax.numpy as jnp
from jax import lax
from jax.experimental import pallas as pl
from jax.experimental.pallas import tpu as pltpu
from jax.experimental.pallas import tpu_sc as plsc

N_GRAPHS = 16
GRID = 8
D = 128
N_NODES = 100000

NW = 32
CHUNK = 128
N_CHUNKS = 25
PER_W = CHUNK * N_CHUNKS
N_PAD = NW * PER_W
N_ALLOC = N_PAD + CHUNK
N_FULL = N_NODES // CHUNK
TAIL = N_NODES - N_FULL * CHUNK
N_ZPAD = N_ALLOC // CHUNK - N_FULL - 1
NBINS = N_GRAPHS * GRID * GRID
ROWS_PER_TILE = NBINS // 16

_mesh = plsc.VectorSubcoreMesh(core_axis_name="c", subcore_axis_name="s")


@functools.partial(
    pl.kernel,
    out_type=jax.ShapeDtypeStruct((2, NBINS, D), jnp.float32),
    mesh=_mesh,
    compiler_params=pltpu.CompilerParams(needs_layout_passes=False),
    scratch_types=[
        pltpu.VMEM((CHUNK, D), jnp.float32),
        pltpu.VMEM((CHUNK, D), jnp.float32),
        pltpu.VMEM((CHUNK, 4), jnp.float32),
        pltpu.VMEM((CHUNK, 4), jnp.float32),
        pltpu.VMEM((CHUNK,), jnp.int32),
        pltpu.VMEM((CHUNK,), jnp.int32),
        pltpu.VMEM((CHUNK,), jnp.float32),
        pltpu.VMEM((ROWS_PER_TILE, D), jnp.float32),
        pltpu.VMEM_SHARED((NBINS, D), jnp.float32),
        pltpu.SemaphoreType.DMA,
        pltpu.SemaphoreType.DMA,
        pltpu.SemaphoreType.DMA,
        pltpu.SemaphoreType.DMA,
    ],
)
def _spp_scatter(feat, tailf, zpad, aux, out, feat_a, feat_b, aux_a, aux_b,
                 idx_a, idx_b, inv_c, bounce, acc, sem_fa, sem_fb, sem_sa,
                 sem_sb):
    cid = lax.axis_index("c")
    sid = lax.axis_index("s")
    w = cid * 16 + sid
    base_w = w * PER_W

    iota16 = lax.broadcasted_iota(jnp.int32, (16,), 0)
    col0 = jnp.full((16,), 0, jnp.int32)
    col1 = jnp.full((16,), 1, jnp.int32)
    col2 = jnp.full((16,), 2, jnp.int32)
    col3 = jnp.full((16,), 3, jnp.int32)
    zero16 = jnp.zeros((16,), jnp.float32)

    def start_fetch(c, featb, auxb, sem):
        g = w * N_CHUNKS + c

        @pl.when(g < N_FULL)
        def _():
            pltpu.async_copy(feat.at[pl.ds(g * CHUNK, CHUNK)], featb, sem)

        @pl.when(g == N_FULL)
        def _():
            pltpu.async_copy(tailf.at[pl.ds(0, CHUNK)], featb, sem)

        @pl.when(g > N_FULL)
        def _():
            pltpu.async_copy(
                zpad.at[pl.ds((g - N_FULL - 1) * CHUNK, CHUNK)], featb, sem)

        pltpu.async_copy(aux.at[pl.ds(g * CHUNK, CHUNK)], auxb, sem)

    def wait_fetch(featb, auxb, sem):
        pltpu.make_async_copy(feat.at[pl.ds(0, CHUNK)], featb, sem).wait()
        pltpu.make_async_copy(aux.at[pl.ds(0, CHUNK)], auxb, sem).wait()

    def process(featb, auxb, idxb):
        def bin_body(i, _):
            rows = iota16 + i * 16
            xv = plsc.load_gather(auxb, [rows, col0])
            yv = plsc.load_gather(auxb, [rows, col1])
            gv = plsc.load_gather(auxb, [rows, col3])
            cv = plsc.load_gather(auxb, [rows, col2])
            binv = gv * float(GRID * GRID) + xv * float(GRID) + yv
            sl = pl.ds(i * 16, 16)
            idxb[sl] = binv.astype(jnp.int32)
            inv_c[sl] = 1.0 / cv
            return 0

        lax.fori_loop(0, CHUNK // 16, bin_body, 0, unroll=2)

        def node_body(j, _):
            inv = plsc.load_gather(inv_c, [jnp.full((16,), j, jnp.int32)])
            for db in range(D // 16):
                sl = pl.ds(db * 16, 16)
                featb[j, sl] = featb[j, sl] * inv
            return 0

        lax.fori_loop(0, CHUNK, node_body, 0, unroll=4)

    def zero_row(r, _):
        for db in range(D // 16):
            bounce[r, pl.ds(db * 16, 16)] = zero16
        return 0

    lax.fori_loop(0, ROWS_PER_TILE, zero_row, 0)
    pltpu.sync_copy(bounce, acc.at[pl.ds(sid * ROWS_PER_TILE, ROWS_PER_TILE)])
    plsc.subcore_barrier()

    start_fetch(0, feat_a, aux_a, sem_fa)
    start_fetch(1, feat_b, aux_b, sem_fb)

    def pipe_body(k, _):
        ca = 2 * k
        wait_fetch(feat_a, aux_a, sem_fa)
        process(feat_a, aux_a, idx_a)
        h_sa = pltpu.async_copy(feat_a, acc.at[idx_a], sem_sa, add=True)
        wait_fetch(feat_b, aux_b, sem_fb)
        process(feat_b, aux_b, idx_b)
        h_sa.wait()
        start_fetch(ca + 2, feat_a, aux_a, sem_fa)
        h_sb = pltpu.async_copy(feat_b, acc.at[idx_b], sem_sb, add=True)
        h_sb.wait()
        start_fetch(ca + 3, feat_b, aux_b, sem_fb)
        return 0

    lax.fori_loop(0, (N_CHUNKS - 1) // 2, pipe_body, 0)

    wait_fetch(feat_a, aux_a, sem_fa)
    process(feat_a, aux_a, idx_a)
    pltpu.async_copy(feat_a, acc.at[idx_a], sem_sa, add=True).wait()
    wait_fetch(feat_b, aux_b, sem_fb)

    plsc.subcore_barrier()
    sl = pl.ds(sid * ROWS_PER_TILE, ROWS_PER_TILE)
    pltpu.sync_copy(acc.at[sl], bounce)
    pltpu.sync_copy(bounce, out.at[cid, sl])


def _add_partials_body(p_ref, o_ref):
    o_ref[...] = p_ref[0] + p_ref[1]


_add_partials = pl.pallas_call(
    _add_partials_body,
    out_shape=jax.ShapeDtypeStruct((NBINS, D), jnp.float32),
)


def kernel(features, xy, graph_ids):
    tailf = jnp.zeros((CHUNK, D), jnp.float32).at[:TAIL].set(
        features[N_FULL * CHUNK:])
    zpad = jnp.zeros((N_ZPAD * CHUNK, D), jnp.float32)
    pad = N_ALLOC - N_NODES
    aux = jnp.concatenate([
        xy[:, :2].astype(jnp.float32),
        xy[:, 2:3].astype(jnp.float32),
        graph_ids[:, None].astype(jnp.float32),
    ], axis=1)
    aux_pad = jnp.tile(
        jnp.array([[0.0, 0.0, 1.0, 0.0]], jnp.float32), (pad, 1))
    auxp = jnp.concatenate([aux, aux_pad], axis=0)
    partials = _spp_scatter(features, tailf, zpad, auxp)
    out = _add_partials(partials)
    return out.reshape(N_GRAPHS, GRID, GRID, D)
